# baseline (speedup 1.0000x reference)
import functools

import jax
import jax.numpy as jnp
from jax import lax
from jax.experimental import pallas as pl
from jax.experimental.pallas import tpu as pltpu
from jax.experimental.pallas import tpu_sc as plsc

MAXL = 8
B, S, D = 8, 2048, 128
NW = 32
Q = 4
QS = S // Q
TR = 128
NT = QS // TR
NIN = QS + MAXL
NG = D // 16

_BASE = []
_off = 0
for _L in range(1, MAXL + 1):
    _BASE.append(_off)
    _off += S - _L
OUT_S = _off

_mesh = plsc.VectorSubcoreMesh(core_axis_name="c", subcore_axis_name="s")


@functools.partial(
    pl.kernel,
    out_type=jax.ShapeDtypeStruct((B * OUT_S, D), jnp.float32),
    mesh=_mesh,
    compiler_params=pltpu.CompilerParams(use_tc_tiling_on_sc=False),
    scratch_types=[
        pltpu.VMEM((NIN, D), jnp.float32),
        pltpu.VMEM((2, TR, D), jnp.float32),
        pltpu.SemaphoreType.DMA,
        pltpu.SemaphoreType.DMA,
        pltpu.SemaphoreType.DMA,
    ],
)
def _span_kernel(t_hbm, out_hbm, pbuf, stage, sem_in, sem0, sem1):
    wid = lax.axis_index("s") * 2 + lax.axis_index("c")
    b = wid // Q
    q = wid % Q
    s0 = q * QS
    ls = pl.multiple_of(jnp.where(q == Q - 1, S - NIN, s0), 8)
    zeros = jnp.zeros((16,), jnp.float32)
    sems = (sem0, sem1)

    pltpu.async_copy(t_hbm.at[pl.ds(b * S + ls, NIN)], pbuf, sem_in).wait()

    for g in range(NG):
        pbuf[0, pl.ds(g * 16, 16)] = zeros

    def pfx(j, c):
        for g in range(NG):
            sl = pl.ds(g * 16, 16)
            pbuf[j, sl] = pbuf[j, sl] + pbuf[j - 1, sl]
        return c

    lax.fori_loop(1, NIN, pfx, 0)

    pending = [None, None]
    k = 0
    for L in range(1, MAXL + 1):
        for t in (NT - 1,) + tuple(range(NT - 1)):
            if t == NT - 1:
                i0 = jnp.where(q == Q - 1, S - L - TR, s0 + t * TR)
            else:
                i0 = s0 + t * TR
            m0 = i0 - ls
            p = k % 2
            k += 1
            if pending[p] is not None:
                pending[p].wait()

            def gbody(g, c, L=L, p=p, m0=m0):
                sl = pl.ds(g * 16, 16)
                ring = tuple(pbuf[m0 + j, sl] for j in range(L))

                def rbody(r, ring):
                    new = pbuf[m0 + r + L, sl]
                    stage[p, r, sl] = new - ring[0]
                    return ring[1:] + (new,)

                lax.fori_loop(0, TR, rbody, ring, unroll=8)
                return c

            lax.fori_loop(0, NG, gbody, 0)
            row0 = b * OUT_S + _BASE[L - 1] + i0
            pending[p] = pltpu.async_copy(
                stage.at[p], out_hbm.at[pl.ds(row0, TR)], sems[p])

    for p in (0, 1):
        if pending[p] is not None:
            pending[p].wait()


def kernel(tensor):
    out = _span_kernel(tensor.reshape(B * S, D))
    return out.reshape(B, OUT_S, D)
